# baseline (speedup 1.0000x reference)
import functools

import jax
import jax.numpy as jnp
from jax import lax
from jax.experimental import pallas as pl
from jax.experimental.pallas import tpu as pltpu
from jax.experimental.pallas import tpu_sc as plsc

_NC = 2
_NS = 16
_NW = _NC * _NS
_L = 16

_C = 128
_SEG = 5


def _make_sc_gather(B, N, D, E, seg_edges, chunk0):
    assert seg_edges % _C == 0
    nch = seg_edges // _C

    mesh = plsc.VectorSubcoreMesh(core_axis_name="c", subcore_axis_name="s")

    @functools.partial(
        pl.kernel,
        mesh=mesh,
        compiler_params=pltpu.CompilerParams(
            use_tc_tiling_on_sc=False, needs_layout_passes=False
        ),
        out_type=(
            jax.ShapeDtypeStruct((seg_edges, D), jnp.float32),
            jax.ShapeDtypeStruct((seg_edges, D), jnp.float32),
            jax.ShapeDtypeStruct((seg_edges,), jnp.float32),
        ),
        scratch_types=(
            pltpu.VMEM((_C,), jnp.int32),
            pltpu.VMEM((_C,), jnp.int32),
            pltpu.VMEM((_C,), jnp.int32),
            pltpu.VMEM((_C,), jnp.int32),
            pltpu.VMEM((_C, D), jnp.float32),
            pltpu.VMEM((_C, D), jnp.float32),
            pltpu.VMEM((_C, 16), jnp.float32),
            pltpu.VMEM((_C, 16), jnp.float32),
            pltpu.VMEM((_C,), jnp.float32),
            pltpu.SemaphoreType.DMA,
        ),
    )
    def sc_gather(ei_hbm, ej_hbm, emb_hbm, locs_hbm,
                  gi_hbm, gj_hbm, d2_hbm,
                  eiv, ejv, idxi, idxj, gi_v, gj_v, li_v, lj_v, d2v, sem):
        wid = lax.axis_index("s") * _NC + lax.axis_index("c")
        nk = (nch - wid + _NW - 1) // _NW

        def chunk_body(k, _):
            c = wid + k * _NW
            base = c * _C
            pltpu.sync_copy(ei_hbm.at[pl.ds(base, _C)], eiv)
            pltpu.sync_copy(ej_hbm.at[pl.ds(base, _C)], ejv)
            gbase = base + chunk0 * _C
            b0 = gbase // E
            rem = gbase - b0 * E
            for t in range(_C // _L):
                goff = rem + t * _L + lax.iota(jnp.int32, _L)
                b = b0 + jnp.where(goff >= E, 1, 0)
                boff = jnp.minimum(b, B - 1) * N
                idxi[pl.ds(t * _L, _L)] = jnp.maximum(eiv[pl.ds(t * _L, _L)], 0) + boff
                idxj[pl.ds(t * _L, _L)] = jnp.maximum(ejv[pl.ds(t * _L, _L)], 0) + boff
            cp1 = pltpu.async_copy(emb_hbm.at[idxi], gi_v, sem)
            cp2 = pltpu.async_copy(emb_hbm.at[idxj], gj_v, sem)
            cp3 = pltpu.async_copy(locs_hbm.at[idxi], li_v, sem)
            cp4 = pltpu.async_copy(locs_hbm.at[idxj], lj_v, sem)
            cp1.wait()
            cp2.wait()
            cp3.wait()
            cp4.wait()
            for t in range(_C // _L):
                r = t * _L + lax.iota(jnp.int32, _L)
                zero = jnp.zeros((_L,), jnp.int32)
                one = zero + 1
                dx = plsc.load_gather(li_v, [r, zero]) - plsc.load_gather(lj_v, [r, zero])
                dy = plsc.load_gather(li_v, [r, one]) - plsc.load_gather(lj_v, [r, one])
                d2v[pl.ds(t * _L, _L)] = dx * dx + dy * dy
            pltpu.sync_copy(gi_v, gi_hbm.at[pl.ds(base, _C)])
            pltpu.sync_copy(gj_v, gj_hbm.at[pl.ds(base, _C)])
            pltpu.sync_copy(d2v, d2_hbm.at[pl.ds(base, _C)])
            return ()

        lax.fori_loop(0, nk, chunk_body, ())

    return sc_gather


def _tc_mlp_body(gi_ref, gj_ref, d2_ref, ei_ref, ej_ref,
                 w1a_ref, w1b_ref, w1d_ref, b1_ref, w2_ref, b2_ref, out_ref):
    h = jnp.dot(gi_ref[...], w1a_ref[...], preferred_element_type=jnp.float32)
    h = h + jnp.dot(gj_ref[...], w1b_ref[...], preferred_element_type=jnp.float32)
    dist = jnp.sqrt(d2_ref[0])
    h = h + dist.T * w1d_ref[...] + b1_ref[...]
    h = jnp.maximum(h, 0.0)
    logits = jnp.sum(h * w2_ref[...], axis=1) + b2_ref[0]
    valid = (ei_ref[0][0] >= 0) & (ej_ref[0][0] >= 0)
    out_ref[0, 0] = jnp.where(valid, logits, -jnp.inf)


def _tc_mlp(gi, gj, d2, ei, ej, W1a, W1b, w1d, b1r, w2r, b2, BLK):
    SEG_E, D = gi.shape
    H = W1a.shape[1]
    nb = SEG_E // BLK
    d2r = d2.reshape(nb, 1, BLK)
    eir = ei.reshape(nb, 1, BLK)
    ejr = ej.reshape(nb, 1, BLK)
    return pl.pallas_call(
        _tc_mlp_body,
        grid=(nb,),
        in_specs=[
            pl.BlockSpec((BLK, D), lambda i: (i, 0)),
            pl.BlockSpec((BLK, D), lambda i: (i, 0)),
            pl.BlockSpec((1, 1, BLK), lambda i: (i, 0, 0)),
            pl.BlockSpec((1, 1, BLK), lambda i: (i, 0, 0)),
            pl.BlockSpec((1, 1, BLK), lambda i: (i, 0, 0)),
            pl.BlockSpec((D, H), lambda i: (0, 0)),
            pl.BlockSpec((D, H), lambda i: (0, 0)),
            pl.BlockSpec((1, H), lambda i: (0, 0)),
            pl.BlockSpec((1, H), lambda i: (0, 0)),
            pl.BlockSpec((1, H), lambda i: (0, 0)),
            pl.BlockSpec(memory_space=pltpu.SMEM),
        ],
        out_specs=pl.BlockSpec((1, 1, BLK), lambda i: (i, 0, 0)),
        out_shape=jax.ShapeDtypeStruct((nb, 1, BLK), jnp.float32),
    )(gi, gj, d2r, eir, ejr, W1a, W1b, w1d, b1r, w2r, b2)


@jax.jit
def kernel(node_embeddings, locs, edge_list, W1, b1, W2, b2):
    B, N, D = node_embeddings.shape
    E = edge_list.shape[1]
    H = W1.shape[1]
    BE = B * E

    emb_flat = node_embeddings.reshape(B * N, D)
    locs_flat = jnp.pad(locs.reshape(B * N, 2), ((0, 0), (0, 14)))
    ei = edge_list[..., 0].reshape(BE)
    ej = edge_list[..., 1].reshape(BE)

    W1a = W1[:D]
    W1b = W1[D:2 * D]
    w1d = W1[2 * D:2 * D + 1]
    b1r = b1.reshape(1, H)
    w2r = W2.reshape(H, 1).T

    nseg = _SEG if BE % (_SEG * _C) == 0 else 1
    seg_edges = BE // nseg
    BLK = next(blk for blk in (2000, 640, 512, 256, 128) if seg_edges % blk == 0)

    gathered = []
    for s in range(nseg):
        lo = s * seg_edges
        ei_s = lax.slice(ei, (lo,), (lo + seg_edges,))
        ej_s = lax.slice(ej, (lo,), (lo + seg_edges,))
        gi, gj, d2 = _make_sc_gather(B, N, D, E, seg_edges, lo // _C)(
            ei_s, ej_s, emb_flat, locs_flat)
        gathered.append((gi, gj, d2, ei_s, ej_s))
    outs = [_tc_mlp(gi, gj, d2, ei_s, ej_s, W1a, W1b, w1d, b1r, w2r, b2, BLK)
            for gi, gj, d2, ei_s, ej_s in gathered]
    out = jnp.concatenate(outs, axis=0) if nseg > 1 else outs[0]
    return out.reshape(B, E)


# speedup vs baseline: 1.0509x; 1.0509x over previous
import functools

import jax
import jax.numpy as jnp
from jax import lax
from jax.experimental import pallas as pl
from jax.experimental.pallas import tpu as pltpu
from jax.experimental.pallas import tpu_sc as plsc

_NC = 2
_NS = 16
_NW = _NC * _NS
_L = 16

_C = 128
_SEG = 5


def _make_sc_gather(B, N, D, E, seg_edges, chunk0):
    assert seg_edges % _C == 0
    nch = seg_edges // _C

    mesh = plsc.VectorSubcoreMesh(core_axis_name="c", subcore_axis_name="s")

    @functools.partial(
        pl.kernel,
        mesh=mesh,
        compiler_params=pltpu.CompilerParams(
            use_tc_tiling_on_sc=False, needs_layout_passes=False
        ),
        out_type=(
            jax.ShapeDtypeStruct((seg_edges, D), jnp.float32),
            jax.ShapeDtypeStruct((seg_edges, D), jnp.float32),
            jax.ShapeDtypeStruct((seg_edges,), jnp.float32),
        ),
        scratch_types=(
            pltpu.VMEM((_C,), jnp.int32),
            pltpu.VMEM((_C,), jnp.int32),
            pltpu.VMEM((_C,), jnp.int32),
            pltpu.VMEM((_C,), jnp.int32),
            pltpu.VMEM((_C,), jnp.int32),
            pltpu.VMEM((_C,), jnp.int32),
            pltpu.VMEM((_C, D), jnp.float32),
            pltpu.VMEM((_C, D), jnp.float32),
            pltpu.VMEM((_C, 16), jnp.float32),
            pltpu.VMEM((_C, 16), jnp.float32),
            pltpu.VMEM((_C,), jnp.float32),
            pltpu.SemaphoreType.DMA,
        ),
    )
    def sc_gather(ei_hbm, ej_hbm, emb_hbm, locs_hbm,
                  gi_hbm, gj_hbm, d2_hbm,
                  eiv, ejv, idxi, idxj, lri, lrj, gi_v, gj_v, li_v, lj_v,
                  d2v, sem):
        wid = lax.axis_index("s") * _NC + lax.axis_index("c")
        nk = (nch - wid + _NW - 1) // _NW

        def chunk_body(k, _):
            c = wid + k * _NW
            base = c * _C
            pltpu.sync_copy(ei_hbm.at[pl.ds(base, _C)], eiv)
            pltpu.sync_copy(ej_hbm.at[pl.ds(base, _C)], ejv)
            gbase = base + chunk0 * _C
            b0 = gbase // E
            rem = gbase - b0 * E
            for t in range(_C // _L):
                goff = rem + t * _L + lax.iota(jnp.int32, _L)
                b = b0 + jnp.where(goff >= E, 1, 0)
                boff = jnp.minimum(b, B - 1) * N
                fi = jnp.maximum(eiv[pl.ds(t * _L, _L)], 0) + boff
                fj = jnp.maximum(ejv[pl.ds(t * _L, _L)], 0) + boff
                idxi[pl.ds(t * _L, _L)] = fi
                idxj[pl.ds(t * _L, _L)] = fj
                lri[pl.ds(t * _L, _L)] = fi >> 3
                lrj[pl.ds(t * _L, _L)] = fj >> 3
            cp1 = pltpu.async_copy(emb_hbm.at[idxi], gi_v, sem)
            cp2 = pltpu.async_copy(emb_hbm.at[idxj], gj_v, sem)
            cp3 = pltpu.async_copy(locs_hbm.at[lri], li_v, sem)
            cp4 = pltpu.async_copy(locs_hbm.at[lrj], lj_v, sem)
            cp1.wait()
            cp2.wait()
            cp3.wait()
            cp4.wait()
            for t in range(_C // _L):
                r = t * _L + lax.iota(jnp.int32, _L)
                lxi = (idxi[pl.ds(t * _L, _L)] & 7) * 2
                lxj = (idxj[pl.ds(t * _L, _L)] & 7) * 2
                one = jnp.zeros((_L,), jnp.int32) + 1
                dx = plsc.load_gather(li_v, [r, lxi]) - plsc.load_gather(lj_v, [r, lxj])
                dy = plsc.load_gather(li_v, [r, lxi + one]) - plsc.load_gather(lj_v, [r, lxj + one])
                d2v[pl.ds(t * _L, _L)] = dx * dx + dy * dy
            pltpu.sync_copy(gi_v, gi_hbm.at[pl.ds(base, _C)])
            pltpu.sync_copy(gj_v, gj_hbm.at[pl.ds(base, _C)])
            pltpu.sync_copy(d2v, d2_hbm.at[pl.ds(base, _C)])
            return ()

        lax.fori_loop(0, nk, chunk_body, ())

    return sc_gather


def _tc_mlp_body(gi_ref, gj_ref, d2_ref, ei_ref, ej_ref,
                 w1a_ref, w1b_ref, w1d_ref, b1_ref, w2_ref, b2_ref, out_ref):
    h = jnp.dot(gi_ref[...], w1a_ref[...], preferred_element_type=jnp.float32)
    h = h + jnp.dot(gj_ref[...], w1b_ref[...], preferred_element_type=jnp.float32)
    dist = jnp.sqrt(d2_ref[0])
    h = h + dist.T * w1d_ref[...] + b1_ref[...]
    h = jnp.maximum(h, 0.0)
    logits = jnp.sum(h * w2_ref[...], axis=1) + b2_ref[0]
    valid = (ei_ref[0][0] >= 0) & (ej_ref[0][0] >= 0)
    out_ref[0, 0] = jnp.where(valid, logits, -jnp.inf)


def _tc_mlp(gi, gj, d2, ei, ej, W1a, W1b, w1d, b1r, w2r, b2, BLK):
    SEG_E, D = gi.shape
    H = W1a.shape[1]
    nb = SEG_E // BLK
    d2r = d2.reshape(nb, 1, BLK)
    eir = ei.reshape(nb, 1, BLK)
    ejr = ej.reshape(nb, 1, BLK)
    return pl.pallas_call(
        _tc_mlp_body,
        grid=(nb,),
        in_specs=[
            pl.BlockSpec((BLK, D), lambda i: (i, 0)),
            pl.BlockSpec((BLK, D), lambda i: (i, 0)),
            pl.BlockSpec((1, 1, BLK), lambda i: (i, 0, 0)),
            pl.BlockSpec((1, 1, BLK), lambda i: (i, 0, 0)),
            pl.BlockSpec((1, 1, BLK), lambda i: (i, 0, 0)),
            pl.BlockSpec((D, H), lambda i: (0, 0)),
            pl.BlockSpec((D, H), lambda i: (0, 0)),
            pl.BlockSpec((1, H), lambda i: (0, 0)),
            pl.BlockSpec((1, H), lambda i: (0, 0)),
            pl.BlockSpec((1, H), lambda i: (0, 0)),
            pl.BlockSpec(memory_space=pltpu.SMEM),
        ],
        out_specs=pl.BlockSpec((1, 1, BLK), lambda i: (i, 0, 0)),
        out_shape=jax.ShapeDtypeStruct((nb, 1, BLK), jnp.float32),
    )(gi, gj, d2r, eir, ejr, W1a, W1b, w1d, b1r, w2r, b2)


@jax.jit
def kernel(node_embeddings, locs, edge_list, W1, b1, W2, b2):
    B, N, D = node_embeddings.shape
    E = edge_list.shape[1]
    H = W1.shape[1]
    BE = B * E

    emb_flat = node_embeddings.reshape(B * N, D)
    assert (B * N) % 8 == 0
    locs_flat = locs.reshape(B * N // 8, 16)
    ei = edge_list[..., 0].reshape(BE)
    ej = edge_list[..., 1].reshape(BE)

    W1a = W1[:D]
    W1b = W1[D:2 * D]
    w1d = W1[2 * D:2 * D + 1]
    b1r = b1.reshape(1, H)
    w2r = W2.reshape(H, 1).T

    nseg = _SEG if BE % (_SEG * _C) == 0 else 1
    seg_edges = BE // nseg
    BLK = next(blk for blk in (2000, 640, 512, 256, 128) if seg_edges % blk == 0)

    gathered = []
    for s in range(nseg):
        lo = s * seg_edges
        ei_s = lax.slice(ei, (lo,), (lo + seg_edges,))
        ej_s = lax.slice(ej, (lo,), (lo + seg_edges,))
        gi, gj, d2 = _make_sc_gather(B, N, D, E, seg_edges, lo // _C)(
            ei_s, ej_s, emb_flat, locs_flat)
        gathered.append((gi, gj, d2, ei_s, ej_s))
    outs = [_tc_mlp(gi, gj, d2, ei_s, ej_s, W1a, W1b, w1d, b1r, w2r, b2, BLK)
            for gi, gj, d2, ei_s, ej_s in gathered]
    out = jnp.concatenate(outs, axis=0) if nseg > 1 else outs[0]
    return out.reshape(B, E)


# speedup vs baseline: 1.0984x; 1.0452x over previous
import functools

import jax
import jax.numpy as jnp
from jax import lax
from jax.experimental import pallas as pl
from jax.experimental.pallas import tpu as pltpu
from jax.experimental.pallas import tpu_sc as plsc

_NC = 2
_NS = 16
_NW = _NC * _NS
_L = 16

_C = 128
_SEG = 5


def _flat_indices(B, N, E, chunk0, c, eiv, ejv, idxi, idxj):
    base = c * _C
    gbase = base + chunk0 * _C
    b0 = gbase // E
    rem = gbase - b0 * E
    for t in range(_C // _L):
        goff = rem + t * _L + lax.iota(jnp.int32, _L)
        b = b0 + jnp.where(goff >= E, 1, 0)
        boff = jnp.minimum(b, B - 1) * N
        idxi[pl.ds(t * _L, _L)] = jnp.maximum(eiv[pl.ds(t * _L, _L)], 0) + boff
        idxj[pl.ds(t * _L, _L)] = jnp.maximum(ejv[pl.ds(t * _L, _L)], 0) + boff


def _make_sc_emb_gather(B, N, D, E, seg_edges, chunk0):
    assert seg_edges % _C == 0
    nch = seg_edges // _C

    mesh = plsc.VectorSubcoreMesh(core_axis_name="c", subcore_axis_name="s")

    @functools.partial(
        pl.kernel,
        mesh=mesh,
        out_type=(
            jax.ShapeDtypeStruct((seg_edges, D), jnp.float32),
            jax.ShapeDtypeStruct((seg_edges, D), jnp.float32),
        ),
        scratch_types=(
            pltpu.VMEM((_C,), jnp.int32),
            pltpu.VMEM((_C,), jnp.int32),
            pltpu.VMEM((_C,), jnp.int32),
            pltpu.VMEM((_C,), jnp.int32),
            pltpu.VMEM((_C, D), jnp.float32),
            pltpu.VMEM((_C, D), jnp.float32),
            pltpu.SemaphoreType.DMA,
        ),
    )
    def sc_emb(ei_hbm, ej_hbm, emb_hbm, gi_hbm, gj_hbm,
               eiv, ejv, idxi, idxj, gi_v, gj_v, sem):
        wid = lax.axis_index("s") * _NC + lax.axis_index("c")
        nk = (nch - wid + _NW - 1) // _NW

        def chunk_body(k, _):
            c = wid + k * _NW
            base = c * _C
            pltpu.sync_copy(ei_hbm.at[pl.ds(base, _C)], eiv)
            pltpu.sync_copy(ej_hbm.at[pl.ds(base, _C)], ejv)
            _flat_indices(B, N, E, chunk0, c, eiv, ejv, idxi, idxj)
            cp1 = pltpu.async_copy(emb_hbm.at[idxi], gi_v, sem)
            cp2 = pltpu.async_copy(emb_hbm.at[idxj], gj_v, sem)
            cp1.wait()
            cp2.wait()
            pltpu.sync_copy(gi_v, gi_hbm.at[pl.ds(base, _C)])
            pltpu.sync_copy(gj_v, gj_hbm.at[pl.ds(base, _C)])
            return ()

        lax.fori_loop(0, nk, chunk_body, ())

    return sc_emb


def _make_sc_d2(B, N, E, seg_edges, chunk0):
    assert seg_edges % _C == 0
    nch = seg_edges // _C

    mesh = plsc.VectorSubcoreMesh(core_axis_name="c", subcore_axis_name="s")

    @functools.partial(
        pl.kernel,
        mesh=mesh,
        compiler_params=pltpu.CompilerParams(
            use_tc_tiling_on_sc=False, needs_layout_passes=False
        ),
        out_type=jax.ShapeDtypeStruct((seg_edges,), jnp.float32),
        scratch_types=(
            pltpu.VMEM((_C,), jnp.int32),
            pltpu.VMEM((_C,), jnp.int32),
            pltpu.VMEM((_C,), jnp.int32),
            pltpu.VMEM((_C,), jnp.int32),
            pltpu.VMEM((_C,), jnp.int32),
            pltpu.VMEM((_C,), jnp.int32),
            pltpu.VMEM((_C, 16), jnp.float32),
            pltpu.VMEM((_C, 16), jnp.float32),
            pltpu.VMEM((_C,), jnp.float32),
            pltpu.SemaphoreType.DMA,
        ),
    )
    def sc_d2(ei_hbm, ej_hbm, locs_hbm, d2_hbm,
              eiv, ejv, idxi, idxj, lri, lrj, li_v, lj_v, d2v, sem):
        wid = lax.axis_index("s") * _NC + lax.axis_index("c")
        nk = (nch - wid + _NW - 1) // _NW

        def chunk_body(k, _):
            c = wid + k * _NW
            base = c * _C
            pltpu.sync_copy(ei_hbm.at[pl.ds(base, _C)], eiv)
            pltpu.sync_copy(ej_hbm.at[pl.ds(base, _C)], ejv)
            _flat_indices(B, N, E, chunk0, c, eiv, ejv, idxi, idxj)
            for t in range(_C // _L):
                lri[pl.ds(t * _L, _L)] = idxi[pl.ds(t * _L, _L)] >> 3
                lrj[pl.ds(t * _L, _L)] = idxj[pl.ds(t * _L, _L)] >> 3
            cp3 = pltpu.async_copy(locs_hbm.at[lri], li_v, sem)
            cp4 = pltpu.async_copy(locs_hbm.at[lrj], lj_v, sem)
            cp3.wait()
            cp4.wait()
            for t in range(_C // _L):
                r = t * _L + lax.iota(jnp.int32, _L)
                lxi = (idxi[pl.ds(t * _L, _L)] & 7) * 2
                lxj = (idxj[pl.ds(t * _L, _L)] & 7) * 2
                one = jnp.zeros((_L,), jnp.int32) + 1
                dx = plsc.load_gather(li_v, [r, lxi]) - plsc.load_gather(lj_v, [r, lxj])
                dy = plsc.load_gather(li_v, [r, lxi + one]) - plsc.load_gather(lj_v, [r, lxj + one])
                d2v[pl.ds(t * _L, _L)] = dx * dx + dy * dy
            pltpu.sync_copy(d2v, d2_hbm.at[pl.ds(base, _C)])
            return ()

        lax.fori_loop(0, nk, chunk_body, ())

    return sc_d2


def _tc_mlp_body(gi_ref, gj_ref, d2_ref, ei_ref, ej_ref,
                 w1a_ref, w1b_ref, w1d_ref, b1_ref, w2_ref, b2_ref, out_ref):
    h = jnp.dot(gi_ref[...], w1a_ref[...], preferred_element_type=jnp.float32)
    h = h + jnp.dot(gj_ref[...], w1b_ref[...], preferred_element_type=jnp.float32)
    dist = jnp.sqrt(d2_ref[0])
    h = h + dist.T * w1d_ref[...] + b1_ref[...]
    h = jnp.maximum(h, 0.0)
    logits = jnp.sum(h * w2_ref[...], axis=1) + b2_ref[0]
    valid = (ei_ref[0][0] >= 0) & (ej_ref[0][0] >= 0)
    out_ref[0, 0] = jnp.where(valid, logits, -jnp.inf)


def _tc_mlp(gi, gj, d2, ei, ej, W1a, W1b, w1d, b1r, w2r, b2, BLK):
    SEG_E, D = gi.shape
    H = W1a.shape[1]
    nb = SEG_E // BLK
    d2r = d2.reshape(nb, 1, BLK)
    eir = ei.reshape(nb, 1, BLK)
    ejr = ej.reshape(nb, 1, BLK)
    return pl.pallas_call(
        _tc_mlp_body,
        grid=(nb,),
        in_specs=[
            pl.BlockSpec((BLK, D), lambda i: (i, 0)),
            pl.BlockSpec((BLK, D), lambda i: (i, 0)),
            pl.BlockSpec((1, 1, BLK), lambda i: (i, 0, 0)),
            pl.BlockSpec((1, 1, BLK), lambda i: (i, 0, 0)),
            pl.BlockSpec((1, 1, BLK), lambda i: (i, 0, 0)),
            pl.BlockSpec((D, H), lambda i: (0, 0)),
            pl.BlockSpec((D, H), lambda i: (0, 0)),
            pl.BlockSpec((1, H), lambda i: (0, 0)),
            pl.BlockSpec((1, H), lambda i: (0, 0)),
            pl.BlockSpec((1, H), lambda i: (0, 0)),
            pl.BlockSpec(memory_space=pltpu.SMEM),
        ],
        out_specs=pl.BlockSpec((1, 1, BLK), lambda i: (i, 0, 0)),
        out_shape=jax.ShapeDtypeStruct((nb, 1, BLK), jnp.float32),
    )(gi, gj, d2r, eir, ejr, W1a, W1b, w1d, b1r, w2r, b2)


@jax.jit
def kernel(node_embeddings, locs, edge_list, W1, b1, W2, b2):
    B, N, D = node_embeddings.shape
    E = edge_list.shape[1]
    H = W1.shape[1]
    BE = B * E

    emb_flat = node_embeddings.reshape(B * N, D)
    assert (B * N) % 8 == 0
    locs_flat = locs.reshape(B * N // 8, 16)
    ei = edge_list[..., 0].reshape(BE)
    ej = edge_list[..., 1].reshape(BE)

    W1a = W1[:D]
    W1b = W1[D:2 * D]
    w1d = W1[2 * D:2 * D + 1]
    b1r = b1.reshape(1, H)
    w2r = W2.reshape(H, 1).T

    nseg = _SEG if BE % (_SEG * _C) == 0 else 1
    seg_edges = BE // nseg
    BLK = next(blk for blk in (2000, 640, 512, 256, 128) if seg_edges % blk == 0)

    outs = []
    for s in range(nseg):
        lo = s * seg_edges
        ei_s = lax.slice(ei, (lo,), (lo + seg_edges,))
        ej_s = lax.slice(ej, (lo,), (lo + seg_edges,))
        gi, gj = _make_sc_emb_gather(B, N, D, E, seg_edges, lo // _C)(
            ei_s, ej_s, emb_flat)
        d2 = _make_sc_d2(B, N, E, seg_edges, lo // _C)(ei_s, ej_s, locs_flat)
        outs.append(_tc_mlp(gi, gj, d2, ei_s, ej_s,
                            W1a, W1b, w1d, b1r, w2r, b2, BLK))
    out = jnp.concatenate(outs, axis=0) if nseg > 1 else outs[0]
    return out.reshape(B, E)


# speedup vs baseline: 1.2227x; 1.1132x over previous
import functools

import jax
import jax.numpy as jnp
from jax import lax
from jax.experimental import pallas as pl
from jax.experimental.pallas import tpu as pltpu
from jax.experimental.pallas import tpu_sc as plsc

_NC = 2
_NS = 16
_NW = _NC * _NS
_L = 16

_C = 128
_SEG = 5


def _flat_indices(B, N, E, chunk0, c, eiv, ejv, idxi, idxj):
    base = c * _C
    gbase = base + chunk0 * _C
    b0 = gbase // E
    rem = gbase - b0 * E
    for t in range(_C // _L):
        goff = rem + t * _L + lax.iota(jnp.int32, _L)
        b = b0 + jnp.where(goff >= E, 1, 0)
        boff = jnp.minimum(b, B - 1) * N
        idxi[pl.ds(t * _L, _L)] = jnp.maximum(eiv[pl.ds(t * _L, _L)], 0) + boff
        idxj[pl.ds(t * _L, _L)] = jnp.maximum(ejv[pl.ds(t * _L, _L)], 0) + boff


def _make_sc_emb_gather(B, N, D, E, seg_edges, chunk0):
    assert seg_edges % _C == 0
    nch = seg_edges // _C

    mesh = plsc.VectorSubcoreMesh(core_axis_name="c", subcore_axis_name="s")

    @functools.partial(
        pl.kernel,
        mesh=mesh,
        out_type=(
            jax.ShapeDtypeStruct((seg_edges, D), jnp.float32),
            jax.ShapeDtypeStruct((seg_edges, D), jnp.float32),
        ),
        scratch_types=(
            pltpu.VMEM((_C,), jnp.int32),
            pltpu.VMEM((_C,), jnp.int32),
            pltpu.VMEM((_C,), jnp.int32),
            pltpu.VMEM((_C,), jnp.int32),
            pltpu.VMEM((_C, D), jnp.float32),
            pltpu.VMEM((_C, D), jnp.float32),
            pltpu.SemaphoreType.DMA,
        ),
    )
    def sc_emb(ei_hbm, ej_hbm, emb_hbm, gi_hbm, gj_hbm,
               eiv, ejv, idxi, idxj, gi_v, gj_v, sem):
        wid = lax.axis_index("s") * _NC + lax.axis_index("c")
        nk = (nch - wid + _NW - 1) // _NW

        def chunk_body(k, _):
            c = wid + k * _NW
            base = c * _C
            pltpu.sync_copy(ei_hbm.at[pl.ds(base, _C)], eiv)
            pltpu.sync_copy(ej_hbm.at[pl.ds(base, _C)], ejv)
            _flat_indices(B, N, E, chunk0, c, eiv, ejv, idxi, idxj)
            cp1 = pltpu.async_copy(emb_hbm.at[idxi], gi_v, sem)
            cp2 = pltpu.async_copy(emb_hbm.at[idxj], gj_v, sem)
            cp1.wait()
            cp2.wait()
            pltpu.sync_copy(gi_v, gi_hbm.at[pl.ds(base, _C)])
            pltpu.sync_copy(gj_v, gj_hbm.at[pl.ds(base, _C)])
            return ()

        lax.fori_loop(0, nk, chunk_body, ())

    return sc_emb


def _make_sc_d2(B, N, E, seg_edges, chunk0):
    assert seg_edges % _C == 0
    nch = seg_edges // _C

    mesh = plsc.VectorSubcoreMesh(core_axis_name="c", subcore_axis_name="s")

    @functools.partial(
        pl.kernel,
        mesh=mesh,
        compiler_params=pltpu.CompilerParams(
            use_tc_tiling_on_sc=False, needs_layout_passes=False
        ),
        out_type=jax.ShapeDtypeStruct((seg_edges,), jnp.float32),
        scratch_types=(
            pltpu.VMEM((_C,), jnp.int32),
            pltpu.VMEM((_C,), jnp.int32),
            pltpu.VMEM((_C,), jnp.int32),
            pltpu.VMEM((_C,), jnp.int32),
            pltpu.VMEM((_C,), jnp.int32),
            pltpu.VMEM((_C,), jnp.int32),
            pltpu.VMEM((_C, 16), jnp.float32),
            pltpu.VMEM((_C, 16), jnp.float32),
            pltpu.VMEM((_C,), jnp.float32),
            pltpu.SemaphoreType.DMA,
        ),
    )
    def sc_d2(ei_hbm, ej_hbm, locs_hbm, d2_hbm,
              eiv, ejv, idxi, idxj, lri, lrj, li_v, lj_v, d2v, sem):
        wid = lax.axis_index("s") * _NC + lax.axis_index("c")
        nk = (nch - wid + _NW - 1) // _NW

        def chunk_body(k, _):
            c = wid + k * _NW
            base = c * _C
            pltpu.sync_copy(ei_hbm.at[pl.ds(base, _C)], eiv)
            pltpu.sync_copy(ej_hbm.at[pl.ds(base, _C)], ejv)
            _flat_indices(B, N, E, chunk0, c, eiv, ejv, idxi, idxj)
            for t in range(_C // _L):
                lri[pl.ds(t * _L, _L)] = idxi[pl.ds(t * _L, _L)] >> 3
                lrj[pl.ds(t * _L, _L)] = idxj[pl.ds(t * _L, _L)] >> 3
            cp3 = pltpu.async_copy(locs_hbm.at[lri], li_v, sem)
            cp4 = pltpu.async_copy(locs_hbm.at[lrj], lj_v, sem)
            cp3.wait()
            cp4.wait()
            for t in range(_C // _L):
                r = t * _L + lax.iota(jnp.int32, _L)
                lxi = idxi[pl.ds(t * _L, _L)] & 7
                lxj = idxj[pl.ds(t * _L, _L)] & 7
                eight = jnp.zeros((_L,), jnp.int32) + 8
                dx = plsc.load_gather(li_v, [r, lxi]) - plsc.load_gather(lj_v, [r, lxj])
                dy = plsc.load_gather(li_v, [r, lxi + eight]) - plsc.load_gather(lj_v, [r, lxj + eight])
                d2v[pl.ds(t * _L, _L)] = dx * dx + dy * dy
            pltpu.sync_copy(d2v, d2_hbm.at[pl.ds(base, _C)])
            return ()

        lax.fori_loop(0, nk, chunk_body, ())

    return sc_d2


def _tc_mlp_body(gi_ref, gj_ref, d2_ref, ei_ref, ej_ref,
                 w1a_ref, w1b_ref, w1d_ref, b1_ref, w2_ref, b2_ref, out_ref):
    h = jnp.dot(gi_ref[...], w1a_ref[...], preferred_element_type=jnp.float32)
    h = h + jnp.dot(gj_ref[...], w1b_ref[...], preferred_element_type=jnp.float32)
    dist = jnp.sqrt(d2_ref[0])
    h = h + dist.T * w1d_ref[...] + b1_ref[...]
    h = jnp.maximum(h, 0.0)
    logits = jnp.sum(h * w2_ref[...], axis=1) + b2_ref[0]
    valid = (ei_ref[0][0] >= 0) & (ej_ref[0][0] >= 0)
    out_ref[0, 0] = jnp.where(valid, logits, -jnp.inf)


def _tc_mlp(gi, gj, d2, ei, ej, W1a, W1b, w1d, b1r, w2r, b2, BLK):
    SEG_E, D = gi.shape
    H = W1a.shape[1]
    nb = SEG_E // BLK
    d2r = d2.reshape(nb, 1, BLK)
    eir = ei.reshape(nb, 1, BLK)
    ejr = ej.reshape(nb, 1, BLK)
    return pl.pallas_call(
        _tc_mlp_body,
        grid=(nb,),
        in_specs=[
            pl.BlockSpec((BLK, D), lambda i: (i, 0)),
            pl.BlockSpec((BLK, D), lambda i: (i, 0)),
            pl.BlockSpec((1, 1, BLK), lambda i: (i, 0, 0)),
            pl.BlockSpec((1, 1, BLK), lambda i: (i, 0, 0)),
            pl.BlockSpec((1, 1, BLK), lambda i: (i, 0, 0)),
            pl.BlockSpec((D, H), lambda i: (0, 0)),
            pl.BlockSpec((D, H), lambda i: (0, 0)),
            pl.BlockSpec((1, H), lambda i: (0, 0)),
            pl.BlockSpec((1, H), lambda i: (0, 0)),
            pl.BlockSpec((1, H), lambda i: (0, 0)),
            pl.BlockSpec(memory_space=pltpu.SMEM),
        ],
        out_specs=pl.BlockSpec((1, 1, BLK), lambda i: (i, 0, 0)),
        out_shape=jax.ShapeDtypeStruct((nb, 1, BLK), jnp.float32),
    )(gi, gj, d2r, eir, ejr, W1a, W1b, w1d, b1r, w2r, b2)


@jax.jit
def kernel(node_embeddings, locs, edge_list, W1, b1, W2, b2):
    B, N, D = node_embeddings.shape
    E = edge_list.shape[1]
    H = W1.shape[1]
    BE = B * E

    emb_flat = node_embeddings.reshape(B * N, D)
    assert (B * N) % 8 == 0
    lt = jnp.transpose(locs, (2, 0, 1)).reshape(2, B * N // 8, 8)
    locs_flat = jnp.concatenate([lt[0], lt[1]], axis=1)
    ei = edge_list[..., 0].reshape(BE)
    ej = edge_list[..., 1].reshape(BE)

    W1a = W1[:D]
    W1b = W1[D:2 * D]
    w1d = W1[2 * D:2 * D + 1]
    b1r = b1.reshape(1, H)
    w2r = W2.reshape(H, 1).T

    nseg = _SEG if BE % (_SEG * _C) == 0 else 1
    seg_edges = BE // nseg
    BLK = next(blk for blk in (2000, 640, 512, 256, 128) if seg_edges % blk == 0)

    outs = []
    for s in range(nseg):
        lo = s * seg_edges
        ei_s = lax.slice(ei, (lo,), (lo + seg_edges,))
        ej_s = lax.slice(ej, (lo,), (lo + seg_edges,))
        gi, gj = _make_sc_emb_gather(B, N, D, E, seg_edges, lo // _C)(
            ei_s, ej_s, emb_flat)
        d2 = _make_sc_d2(B, N, E, seg_edges, lo // _C)(ei_s, ej_s, locs_flat)
        outs.append(_tc_mlp(gi, gj, d2, ei_s, ej_s,
                            W1a, W1b, w1d, b1r, w2r, b2, BLK))
    out = jnp.concatenate(outs, axis=0) if nseg > 1 else outs[0]
    return out.reshape(B, E)


# speedup vs baseline: 1.2230x; 1.0003x over previous
import functools

import jax
import jax.numpy as jnp
from jax import lax
from jax.experimental import pallas as pl
from jax.experimental.pallas import tpu as pltpu
from jax.experimental.pallas import tpu_sc as plsc

_NC = 2
_NS = 16
_NW = _NC * _NS
_L = 16

_C = 128
_SEG = 5


def _flat_indices(B, N, E, chunk0, c, eiv, ejv, idxi, idxj):
    base = c * _C
    gbase = base + chunk0 * _C
    b0 = gbase // E
    rem = gbase - b0 * E
    for t in range(_C // _L):
        goff = rem + t * _L + lax.iota(jnp.int32, _L)
        b = b0 + jnp.where(goff >= E, 1, 0)
        boff = jnp.minimum(b, B - 1) * N
        idxi[pl.ds(t * _L, _L)] = jnp.maximum(eiv[pl.ds(t * _L, _L)], 0) + boff
        idxj[pl.ds(t * _L, _L)] = jnp.maximum(ejv[pl.ds(t * _L, _L)], 0) + boff


def _make_sc_emb_gather(B, N, D, E, seg_edges, chunk0):
    assert seg_edges % _C == 0
    nch = seg_edges // _C

    mesh = plsc.VectorSubcoreMesh(core_axis_name="c", subcore_axis_name="s")

    @functools.partial(
        pl.kernel,
        mesh=mesh,
        out_type=(
            jax.ShapeDtypeStruct((seg_edges, D), jnp.float32),
            jax.ShapeDtypeStruct((seg_edges, D), jnp.float32),
        ),
        scratch_types=(
            pltpu.VMEM((_C,), jnp.int32),
            pltpu.VMEM((_C,), jnp.int32),
            pltpu.VMEM((_C,), jnp.int32),
            pltpu.VMEM((_C,), jnp.int32),
            pltpu.VMEM((_C, D), jnp.float32),
            pltpu.VMEM((_C, D), jnp.float32),
            pltpu.SemaphoreType.DMA,
        ),
    )
    def sc_emb(ei_hbm, ej_hbm, emb_hbm, gi_hbm, gj_hbm,
               eiv, ejv, idxi, idxj, gi_v, gj_v, sem):
        wid = lax.axis_index("s") * _NC + lax.axis_index("c")
        nk = (nch - wid + _NW - 1) // _NW

        def chunk_body(k, _):
            c = wid + k * _NW
            base = c * _C
            pltpu.sync_copy(ei_hbm.at[pl.ds(base, _C)], eiv)
            pltpu.sync_copy(ej_hbm.at[pl.ds(base, _C)], ejv)
            _flat_indices(B, N, E, chunk0, c, eiv, ejv, idxi, idxj)
            cp1 = pltpu.async_copy(emb_hbm.at[idxi], gi_v, sem)
            cp2 = pltpu.async_copy(emb_hbm.at[idxj], gj_v, sem)
            cp1.wait()
            cp2.wait()
            pltpu.sync_copy(gi_v, gi_hbm.at[pl.ds(base, _C)])
            pltpu.sync_copy(gj_v, gj_hbm.at[pl.ds(base, _C)])
            return ()

        lax.fori_loop(0, nk, chunk_body, ())

    return sc_emb


def _make_sc_d2(B, N, E, seg_edges, chunk0):
    assert seg_edges % _C == 0
    nch = seg_edges // _C

    mesh = plsc.VectorSubcoreMesh(core_axis_name="c", subcore_axis_name="s")

    @functools.partial(
        pl.kernel,
        mesh=mesh,
        compiler_params=pltpu.CompilerParams(
            use_tc_tiling_on_sc=False, needs_layout_passes=False
        ),
        out_type=jax.ShapeDtypeStruct((seg_edges,), jnp.float32),
        scratch_types=(
            pltpu.VMEM((_C,), jnp.int32),
            pltpu.VMEM((_C,), jnp.int32),
            pltpu.VMEM((_C,), jnp.int32),
            pltpu.VMEM((_C,), jnp.int32),
            pltpu.VMEM((_C,), jnp.int32),
            pltpu.VMEM((_C,), jnp.int32),
            pltpu.VMEM((_C, 16), jnp.float32),
            pltpu.VMEM((_C, 16), jnp.float32),
            pltpu.VMEM((_C,), jnp.float32),
            pltpu.SemaphoreType.DMA,
        ),
    )
    def sc_d2(ei_hbm, ej_hbm, locs_hbm, d2_hbm,
              eiv, ejv, idxi, idxj, lri, lrj, li_v, lj_v, d2v, sem):
        wid = lax.axis_index("s") * _NC + lax.axis_index("c")
        nk = (nch - wid + _NW - 1) // _NW

        def chunk_body(k, _):
            c = wid + k * _NW
            base = c * _C
            pltpu.sync_copy(ei_hbm.at[pl.ds(base, _C)], eiv)
            pltpu.sync_copy(ej_hbm.at[pl.ds(base, _C)], ejv)
            _flat_indices(B, N, E, chunk0, c, eiv, ejv, idxi, idxj)
            for t in range(_C // _L):
                lri[pl.ds(t * _L, _L)] = idxi[pl.ds(t * _L, _L)] >> 3
                lrj[pl.ds(t * _L, _L)] = idxj[pl.ds(t * _L, _L)] >> 3
            cp3 = pltpu.async_copy(locs_hbm.at[lri], li_v, sem)
            cp4 = pltpu.async_copy(locs_hbm.at[lrj], lj_v, sem)
            cp3.wait()
            cp4.wait()
            for t in range(_C // _L):
                r = t * _L + lax.iota(jnp.int32, _L)
                lxi = idxi[pl.ds(t * _L, _L)] & 7
                lxj = idxj[pl.ds(t * _L, _L)] & 7
                eight = jnp.zeros((_L,), jnp.int32) + 8
                dx = plsc.load_gather(li_v, [r, lxi]) - plsc.load_gather(lj_v, [r, lxj])
                dy = plsc.load_gather(li_v, [r, lxi + eight]) - plsc.load_gather(lj_v, [r, lxj + eight])
                d2v[pl.ds(t * _L, _L)] = dx * dx + dy * dy
            pltpu.sync_copy(d2v, d2_hbm.at[pl.ds(base, _C)])
            return ()

        lax.fori_loop(0, nk, chunk_body, ())

    return sc_d2


def _tc_mlp_body(gi_ref, gj_ref, d2_ref, ei_ref, ej_ref,
                 w1a_ref, w1b_ref, w1d_ref, b1_ref, w2_ref, b2_ref, out_ref):
    h = jnp.dot(gi_ref[...], w1a_ref[...], preferred_element_type=jnp.float32)
    h = h + jnp.dot(gj_ref[...], w1b_ref[...], preferred_element_type=jnp.float32)
    dist = jnp.sqrt(d2_ref[0])
    h = h + dist.T * w1d_ref[...] + b1_ref[...]
    h = jnp.maximum(h, 0.0)
    logits = jnp.sum(h * w2_ref[...], axis=1) + b2_ref[0]
    valid = (ei_ref[0][0] >= 0) & (ej_ref[0][0] >= 0)
    out_ref[0, 0] = jnp.where(valid, logits, -jnp.inf)


def _tc_mlp(gi, gj, d2, ei, ej, W1a, W1b, w1d, b1r, w2r, b2, BLK):
    SEG_E, D = gi.shape
    H = W1a.shape[1]
    nb = SEG_E // BLK
    d2r = d2.reshape(nb, 1, BLK)
    eir = ei.reshape(nb, 1, BLK)
    ejr = ej.reshape(nb, 1, BLK)
    return pl.pallas_call(
        _tc_mlp_body,
        grid=(nb,),
        in_specs=[
            pl.BlockSpec((BLK, D), lambda i: (i, 0)),
            pl.BlockSpec((BLK, D), lambda i: (i, 0)),
            pl.BlockSpec((1, 1, BLK), lambda i: (i, 0, 0)),
            pl.BlockSpec((1, 1, BLK), lambda i: (i, 0, 0)),
            pl.BlockSpec((1, 1, BLK), lambda i: (i, 0, 0)),
            pl.BlockSpec((D, H), lambda i: (0, 0)),
            pl.BlockSpec((D, H), lambda i: (0, 0)),
            pl.BlockSpec((1, H), lambda i: (0, 0)),
            pl.BlockSpec((1, H), lambda i: (0, 0)),
            pl.BlockSpec((1, H), lambda i: (0, 0)),
            pl.BlockSpec(memory_space=pltpu.SMEM),
        ],
        out_specs=pl.BlockSpec((1, 1, BLK), lambda i: (i, 0, 0)),
        out_shape=jax.ShapeDtypeStruct((nb, 1, BLK), jnp.float32),
    )(gi, gj, d2r, eir, ejr, W1a, W1b, w1d, b1r, w2r, b2)


@jax.jit
def kernel(node_embeddings, locs, edge_list, W1, b1, W2, b2):
    B, N, D = node_embeddings.shape
    E = edge_list.shape[1]
    H = W1.shape[1]
    BE = B * E

    emb_flat = node_embeddings.reshape(B * N, D)
    assert (B * N) % 8 == 0
    lt = jnp.transpose(locs, (2, 0, 1)).reshape(2, B * N // 8, 8)
    locs_flat = jnp.concatenate([lt[0], lt[1]], axis=1)
    et = jnp.transpose(edge_list, (2, 0, 1))
    ei = et[0].reshape(BE)
    ej = et[1].reshape(BE)

    W1a = W1[:D]
    W1b = W1[D:2 * D]
    w1d = W1[2 * D:2 * D + 1]
    b1r = b1.reshape(1, H)
    w2r = W2.reshape(H, 1).T

    nseg = _SEG if BE % (_SEG * _C) == 0 else 1
    seg_edges = BE // nseg
    BLK = next(blk for blk in (2000, 640, 512, 256, 128) if seg_edges % blk == 0)

    outs = []
    for s in range(nseg):
        lo = s * seg_edges
        ei_s = lax.slice(ei, (lo,), (lo + seg_edges,))
        ej_s = lax.slice(ej, (lo,), (lo + seg_edges,))
        gi, gj = _make_sc_emb_gather(B, N, D, E, seg_edges, lo // _C)(
            ei_s, ej_s, emb_flat)
        d2 = _make_sc_d2(B, N, E, seg_edges, lo // _C)(ei_s, ej_s, locs_flat)
        outs.append(_tc_mlp(gi, gj, d2, ei_s, ej_s,
                            W1a, W1b, w1d, b1r, w2r, b2, BLK))
    out = jnp.concatenate(outs, axis=0) if nseg > 1 else outs[0]
    return out.reshape(B, E)


# speedup vs baseline: 1.3099x; 1.0710x over previous
import functools

import jax
import jax.numpy as jnp
from jax import lax
from jax.experimental import pallas as pl
from jax.experimental.pallas import tpu as pltpu
from jax.experimental.pallas import tpu_sc as plsc

_NC = 2
_NS = 16
_NW = _NC * _NS
_L = 16

_C = 128
_SEG = 5


def _flat_indices(B, N, E, chunk0, c, eiv, ejv, idxi, idxj):
    base = c * _C
    gbase = base + chunk0 * _C
    b0 = gbase // E
    rem = gbase - b0 * E
    for t in range(_C // _L):
        goff = rem + t * _L + lax.iota(jnp.int32, _L)
        b = b0 + jnp.where(goff >= E, 1, 0)
        boff = jnp.minimum(b, B - 1) * N
        idxi[pl.ds(t * _L, _L)] = jnp.maximum(eiv[pl.ds(t * _L, _L)], 0) + boff
        idxj[pl.ds(t * _L, _L)] = jnp.maximum(ejv[pl.ds(t * _L, _L)], 0) + boff


def _make_sc_emb_gather(B, N, D, E, seg_edges, chunk0):
    assert seg_edges % _C == 0
    nch = seg_edges // _C

    mesh = plsc.VectorSubcoreMesh(core_axis_name="c", subcore_axis_name="s")

    @functools.partial(
        pl.kernel,
        mesh=mesh,
        out_type=(
            jax.ShapeDtypeStruct((seg_edges, D), jnp.float32),
            jax.ShapeDtypeStruct((seg_edges, D), jnp.float32),
        ),
        scratch_types=(
            pltpu.VMEM((_C,), jnp.int32),
            pltpu.VMEM((_C,), jnp.int32),
            pltpu.VMEM((_C,), jnp.int32),
            pltpu.VMEM((_C,), jnp.int32),
            pltpu.VMEM((2, _C, D), jnp.float32),
            pltpu.VMEM((2, _C, D), jnp.float32),
            pltpu.SemaphoreType.DMA,
            pltpu.SemaphoreType.DMA,
            pltpu.SemaphoreType.DMA,
            pltpu.SemaphoreType.DMA,
            pltpu.SemaphoreType.DMA,
        ),
    )
    def sc_emb(ei_hbm, ej_hbm, emb_hbm, gi_hbm, gj_hbm,
               eiv, ejv, idxi, idxj, gi_v, gj_v, gsem,
               wgi0, wgj0, wgi1, wgj1):
        wid = lax.axis_index("s") * _NC + lax.axis_index("c")
        nk = (nch - wid + _NW - 1) // _NW
        wsems = ((wgi0, wgj0), (wgi1, wgj1))

        def drain_writes(b, base):
            pltpu.make_async_copy(gi_v.at[b], gi_hbm.at[pl.ds(base, _C)],
                                  wsems[b][0]).wait()
            pltpu.make_async_copy(gj_v.at[b], gj_hbm.at[pl.ds(base, _C)],
                                  wsems[b][1]).wait()

        def half(k, b):
            m = 2 * k + b
            c = wid + m * _NW

            @pl.when(c < nch)
            def _():
                base = c * _C
                @pl.when(k >= 1)
                def _():
                    drain_writes(b, base)
                pltpu.sync_copy(ei_hbm.at[pl.ds(base, _C)], eiv)
                pltpu.sync_copy(ej_hbm.at[pl.ds(base, _C)], ejv)
                _flat_indices(B, N, E, chunk0, c, eiv, ejv, idxi, idxj)
                cp1 = pltpu.async_copy(emb_hbm.at[idxi], gi_v.at[b], gsem)
                cp2 = pltpu.async_copy(emb_hbm.at[idxj], gj_v.at[b], gsem)
                cp1.wait()
                cp2.wait()
                pltpu.async_copy(gi_v.at[b], gi_hbm.at[pl.ds(base, _C)],
                                 wsems[b][0])
                pltpu.async_copy(gj_v.at[b], gj_hbm.at[pl.ds(base, _C)],
                                 wsems[b][1])

        def pair_body(k, _):
            half(k, 0)
            half(k, 1)
            return ()

        npairs = (nch // _NW + 2) // 2
        lax.fori_loop(0, npairs, pair_body, ())

        def drain_slot(m):
            for b in (0, 1):
                @pl.when((m & 1) == b)
                def _():
                    drain_writes(b, (wid + m * _NW) * _C)

        @pl.when(nk >= 2)
        def _():
            drain_slot(nk - 2)

        @pl.when(nk >= 1)
        def _():
            drain_slot(nk - 1)

    return sc_emb


def _make_sc_d2(B, N, E, seg_edges, chunk0):
    assert seg_edges % _C == 0
    nch = seg_edges // _C

    mesh = plsc.VectorSubcoreMesh(core_axis_name="c", subcore_axis_name="s")

    @functools.partial(
        pl.kernel,
        mesh=mesh,
        compiler_params=pltpu.CompilerParams(
            use_tc_tiling_on_sc=False, needs_layout_passes=False
        ),
        out_type=jax.ShapeDtypeStruct((seg_edges,), jnp.float32),
        scratch_types=(
            pltpu.VMEM((_C,), jnp.int32),
            pltpu.VMEM((_C,), jnp.int32),
            pltpu.VMEM((_C,), jnp.int32),
            pltpu.VMEM((_C,), jnp.int32),
            pltpu.VMEM((_C,), jnp.int32),
            pltpu.VMEM((_C,), jnp.int32),
            pltpu.VMEM((_C, 16), jnp.float32),
            pltpu.VMEM((_C, 16), jnp.float32),
            pltpu.VMEM((_C,), jnp.float32),
            pltpu.SemaphoreType.DMA,
        ),
    )
    def sc_d2(ei_hbm, ej_hbm, locs_hbm, d2_hbm,
              eiv, ejv, idxi, idxj, lri, lrj, li_v, lj_v, d2v, sem):
        wid = lax.axis_index("s") * _NC + lax.axis_index("c")
        nk = (nch - wid + _NW - 1) // _NW

        def chunk_body(k, _):
            c = wid + k * _NW
            base = c * _C
            pltpu.sync_copy(ei_hbm.at[pl.ds(base, _C)], eiv)
            pltpu.sync_copy(ej_hbm.at[pl.ds(base, _C)], ejv)
            _flat_indices(B, N, E, chunk0, c, eiv, ejv, idxi, idxj)
            for t in range(_C // _L):
                lri[pl.ds(t * _L, _L)] = idxi[pl.ds(t * _L, _L)] >> 3
                lrj[pl.ds(t * _L, _L)] = idxj[pl.ds(t * _L, _L)] >> 3
            cp3 = pltpu.async_copy(locs_hbm.at[lri], li_v, sem)
            cp4 = pltpu.async_copy(locs_hbm.at[lrj], lj_v, sem)
            cp3.wait()
            cp4.wait()
            for t in range(_C // _L):
                r = t * _L + lax.iota(jnp.int32, _L)
                lxi = idxi[pl.ds(t * _L, _L)] & 7
                lxj = idxj[pl.ds(t * _L, _L)] & 7
                eight = jnp.zeros((_L,), jnp.int32) + 8
                dx = plsc.load_gather(li_v, [r, lxi]) - plsc.load_gather(lj_v, [r, lxj])
                dy = plsc.load_gather(li_v, [r, lxi + eight]) - plsc.load_gather(lj_v, [r, lxj + eight])
                d2v[pl.ds(t * _L, _L)] = dx * dx + dy * dy
            pltpu.sync_copy(d2v, d2_hbm.at[pl.ds(base, _C)])
            return ()

        lax.fori_loop(0, nk, chunk_body, ())

    return sc_d2


def _tc_mlp_body(gi_ref, gj_ref, d2_ref, ei_ref, ej_ref,
                 w1a_ref, w1b_ref, w1d_ref, b1_ref, w2_ref, b2_ref, out_ref):
    h = jnp.dot(gi_ref[...], w1a_ref[...], preferred_element_type=jnp.float32)
    h = h + jnp.dot(gj_ref[...], w1b_ref[...], preferred_element_type=jnp.float32)
    dist = jnp.sqrt(d2_ref[0])
    h = h + dist.T * w1d_ref[...] + b1_ref[...]
    h = jnp.maximum(h, 0.0)
    logits = jnp.sum(h * w2_ref[...], axis=1) + b2_ref[0]
    valid = (ei_ref[0][0] >= 0) & (ej_ref[0][0] >= 0)
    out_ref[0, 0] = jnp.where(valid, logits, -jnp.inf)


def _tc_mlp(gi, gj, d2, ei, ej, W1a, W1b, w1d, b1r, w2r, b2, BLK):
    SEG_E, D = gi.shape
    H = W1a.shape[1]
    nb = SEG_E // BLK
    d2r = d2.reshape(nb, 1, BLK)
    eir = ei.reshape(nb, 1, BLK)
    ejr = ej.reshape(nb, 1, BLK)
    return pl.pallas_call(
        _tc_mlp_body,
        grid=(nb,),
        in_specs=[
            pl.BlockSpec((BLK, D), lambda i: (i, 0)),
            pl.BlockSpec((BLK, D), lambda i: (i, 0)),
            pl.BlockSpec((1, 1, BLK), lambda i: (i, 0, 0)),
            pl.BlockSpec((1, 1, BLK), lambda i: (i, 0, 0)),
            pl.BlockSpec((1, 1, BLK), lambda i: (i, 0, 0)),
            pl.BlockSpec((D, H), lambda i: (0, 0)),
            pl.BlockSpec((D, H), lambda i: (0, 0)),
            pl.BlockSpec((1, H), lambda i: (0, 0)),
            pl.BlockSpec((1, H), lambda i: (0, 0)),
            pl.BlockSpec((1, H), lambda i: (0, 0)),
            pl.BlockSpec(memory_space=pltpu.SMEM),
        ],
        out_specs=pl.BlockSpec((1, 1, BLK), lambda i: (i, 0, 0)),
        out_shape=jax.ShapeDtypeStruct((nb, 1, BLK), jnp.float32),
    )(gi, gj, d2r, eir, ejr, W1a, W1b, w1d, b1r, w2r, b2)


@jax.jit
def kernel(node_embeddings, locs, edge_list, W1, b1, W2, b2):
    B, N, D = node_embeddings.shape
    E = edge_list.shape[1]
    H = W1.shape[1]
    BE = B * E

    emb_flat = node_embeddings.reshape(B * N, D)
    assert (B * N) % 8 == 0
    lt = jnp.transpose(locs, (2, 0, 1)).reshape(2, B * N // 8, 8)
    locs_flat = jnp.concatenate([lt[0], lt[1]], axis=1)
    et = jnp.transpose(edge_list, (2, 0, 1))
    ei = et[0].reshape(BE)
    ej = et[1].reshape(BE)

    W1a = W1[:D]
    W1b = W1[D:2 * D]
    w1d = W1[2 * D:2 * D + 1]
    b1r = b1.reshape(1, H)
    w2r = W2.reshape(H, 1).T

    nseg = _SEG if BE % (_SEG * _C) == 0 else 1
    seg_edges = BE // nseg
    BLK = next(blk for blk in (2000, 640, 512, 256, 128) if seg_edges % blk == 0)

    outs = []
    for s in range(nseg):
        lo = s * seg_edges
        ei_s = lax.slice(ei, (lo,), (lo + seg_edges,))
        ej_s = lax.slice(ej, (lo,), (lo + seg_edges,))
        gi, gj = _make_sc_emb_gather(B, N, D, E, seg_edges, lo // _C)(
            ei_s, ej_s, emb_flat)
        d2 = _make_sc_d2(B, N, E, seg_edges, lo // _C)(ei_s, ej_s, locs_flat)
        outs.append(_tc_mlp(gi, gj, d2, ei_s, ej_s,
                            W1a, W1b, w1d, b1r, w2r, b2, BLK))
    out = jnp.concatenate(outs, axis=0) if nseg > 1 else outs[0]
    return out.reshape(B, E)
